# Initial kernel scaffold; baseline (speedup 1.0000x reference)
import jax
import jax.numpy as jnp
from jax.experimental import pallas as pl


def kernel(x, edge_index, W, attn_l, attn_r):
    raise NotImplementedError("write your pallas kernel here")


# speedup vs baseline: 22.3871x; 22.3871x over previous
import jax
import jax.numpy as jnp
from jax import lax
from jax.experimental import pallas as pl
from jax.experimental.pallas import tpu as pltpu
from jax.experimental.pallas import tpu_sc as plsc

N = 10000
E = 320000
D = 128
OUT = 128

CHUNK = 128
NCHUNKS = E // CHUNK
NC, NS = 2, 16
NW = NC * NS
CPW = -(-NCHUNKS // NW)
NPAD = CPW * CHUNK
SEG = NPAD // NS

_mesh = plsc.VectorSubcoreMesh(
    core_axis_name="c", subcore_axis_name="s", num_cores=NC, num_subcores=NS
)
_sc_params = pltpu.CompilerParams(needs_layout_passes=False)


def _zero_vec(ref, nwords):
  def body(i, _):
    ref[pl.ds(i * 16, 16)] = jnp.zeros((16,), jnp.float32)
    return 0
  lax.fori_loop(0, nwords // 16, body, 0)


def _deg_body(edges, deg_hbm, ind_hbm, idx_v, ones_v, zrow_v, deg_sp,
              ind_sp, sem):
  c = lax.axis_index("c")
  s = lax.axis_index("s")
  wid = s * NC + c

  for k in range(CHUNK // 16):
    ones_v[0, pl.ds(k * 16, 16)] = jnp.ones((16,), jnp.float32)
  _zero_vec(zrow_v, 640)
  pltpu.sync_copy(zrow_v.at[pl.ds(0, SEG)], deg_sp.at[pl.ds(s * SEG, SEG)])
  pltpu.sync_copy(zrow_v.at[pl.ds(0, SEG)], ind_sp.at[pl.ds(s * SEG, SEG)])
  plsc.subcore_barrier()

  def chunk_body(k, _):
    cidx = k * NW + wid

    @pl.when(cidx < NCHUNKS)
    def _():
      pltpu.async_copy(
          edges.at[:, pl.ds(cidx * CHUNK, CHUNK)], idx_v.at[0], sem
      ).wait()
      pltpu.sync_copy(ones_v.at[0], deg_sp.at[idx_v.at[0, 0]], add=True)
      pltpu.sync_copy(ones_v.at[0], ind_sp.at[idx_v.at[0, 1]], add=True)

    return 0

  lax.fori_loop(0, CPW, chunk_body, 0)
  plsc.subcore_barrier()
  pltpu.sync_copy(deg_sp.at[pl.ds(s * SEG, SEG)], zrow_v.at[pl.ds(0, SEG)])
  pltpu.sync_copy(
      zrow_v.at[pl.ds(0, SEG)],
      deg_hbm.at[pl.ds(c * NPAD + s * SEG, SEG)],
  )
  pltpu.sync_copy(ind_sp.at[pl.ds(s * SEG, SEG)], zrow_v.at[pl.ds(0, SEG)])
  pltpu.sync_copy(
      zrow_v.at[pl.ds(0, SEG)],
      ind_hbm.at[pl.ds(c * NPAD + s * SEG, SEG)],
  )


_deg_call = pl.kernel(
    _deg_body,
    out_type=(
        jax.ShapeDtypeStruct((NC * NPAD,), jnp.float32),
        jax.ShapeDtypeStruct((NC * NPAD,), jnp.float32),
    ),
    mesh=_mesh,
    compiler_params=_sc_params,
    scratch_types=[
        pltpu.VMEM((1, 2, CHUNK), jnp.int32),
        pltpu.VMEM((1, CHUNK), jnp.float32),
        pltpu.VMEM((640,), jnp.float32),
        pltpu.VMEM_SHARED((NPAD,), jnp.float32),
        pltpu.VMEM_SHARED((NPAD,), jnp.float32),
        pltpu.SemaphoreType.DMA,
    ],
)


def _proj_body(x_ref, w_ref, al_ref, ar_ref, deg_ref, feat_ref, elr_ref):
  feat = jnp.dot(x_ref[...], w_ref[...], preferred_element_type=jnp.float32)
  deg = jnp.maximum(deg_ref[0] + deg_ref[1], 1.0)
  so = lax.rsqrt(deg)
  fs = feat * so[:, None]
  feat_ref[...] = fs
  el = lax.dot_general(
      al_ref[...], fs, (((1,), (1,)), ((), ())),
      preferred_element_type=jnp.float32,
  )
  er = lax.dot_general(
      ar_ref[...], feat, (((1,), (1,)), ((), ())),
      preferred_element_type=jnp.float32,
  )
  elr_ref[...] = jnp.concatenate([el, er], axis=0)


def _edge_body(edges, feat_hbm, elr_hbm, acc_hbm, den_hbm,
               tbl_v, idx_v, ee_v, rows_v, zrow_v,
               acc_sp, den_sp, sem):
  c = lax.axis_index("c")
  s = lax.axis_index("s")
  wid = s * NC + c

  pltpu.sync_copy(elr_hbm, tbl_v)

  _zero_vec(zrow_v, 640)

  def zb(r, _):
    for cc in range(D // 16):
      rows_v[0, r, pl.ds(cc * 16, 16)] = jnp.zeros((16,), jnp.float32)
    return 0
  lax.fori_loop(0, CHUNK, zb, 0)

  base = s * SEG
  for k in range(4):
    pltpu.sync_copy(rows_v.at[0], acc_sp.at[pl.ds(base + k * CHUNK, CHUNK)])
  tail = SEG - 4 * CHUNK
  pltpu.sync_copy(
      rows_v.at[0, pl.ds(0, tail)],
      acc_sp.at[pl.ds(base + 4 * CHUNK, tail)],
  )
  pltpu.sync_copy(zrow_v.at[pl.ds(0, SEG)], den_sp.at[pl.ds(s * SEG, SEG)])
  plsc.subcore_barrier()

  def chunk_body(k, _):
    cidx = k * NW + wid

    @pl.when(cidx < NCHUNKS)
    def _():
      pltpu.async_copy(
          edges.at[:, pl.ds(cidx * CHUNK, CHUNK)], idx_v.at[0], sem
      ).wait()
      for v in range(CHUNK // 16):
        si = idx_v[0, 0, pl.ds(v * 16, 16)]
        di = idx_v[0, 1, pl.ds(v * 16, 16)] + NPAD
        e = plsc.load_gather(tbl_v, [si]) + plsc.load_gather(tbl_v, [di])
        e = jnp.where(e > 0, e, 0.2 * e)
        ee_v[0, pl.ds(v * 16, 16)] = jnp.exp(e)
      pltpu.async_copy(feat_hbm.at[idx_v.at[0, 0]], rows_v.at[0], sem).wait()
      def srow(r, _):
        sc = plsc.load_gather(ee_v.at[0], [jnp.zeros((16,), jnp.int32) + r])
        for cc in range(D // 16):
          rows_v[0, r, pl.ds(cc * 16, 16)] = (
              rows_v[0, r, pl.ds(cc * 16, 16)] * sc
          )
        return 0
      lax.fori_loop(0, CHUNK, srow, 0)

      pltpu.sync_copy(rows_v.at[0], acc_sp.at[idx_v.at[0, 1]], add=True)
      pltpu.sync_copy(ee_v.at[0], den_sp.at[idx_v.at[0, 1]], add=True)

    return 0

  lax.fori_loop(0, CPW, chunk_body, 0)
  plsc.subcore_barrier()

  for k in range(4):
    pltpu.sync_copy(acc_sp.at[pl.ds(base + k * CHUNK, CHUNK)], rows_v.at[0])
    pltpu.sync_copy(
        rows_v.at[0], acc_hbm.at[c, pl.ds(base + k * CHUNK, CHUNK)]
    )
  pltpu.sync_copy(
      acc_sp.at[pl.ds(base + 4 * CHUNK, tail)], rows_v.at[0, pl.ds(0, tail)]
  )
  pltpu.sync_copy(
      rows_v.at[0, pl.ds(0, tail)],
      acc_hbm.at[c, pl.ds(base + 4 * CHUNK, tail)],
  )
  pltpu.sync_copy(den_sp.at[pl.ds(s * SEG, SEG)], zrow_v.at[pl.ds(0, SEG)])
  pltpu.sync_copy(
      zrow_v.at[pl.ds(0, SEG)],
      den_hbm.at[pl.ds(c * NPAD + s * SEG, SEG)],
  )


_edge_call = pl.kernel(
    _edge_body,
    out_type=(
        jax.ShapeDtypeStruct((NC, NPAD, D), jnp.float32),
        jax.ShapeDtypeStruct((NC * NPAD,), jnp.float32),
    ),
    mesh=_mesh,
    compiler_params=_sc_params,
    scratch_types=[
        pltpu.VMEM((2 * NPAD,), jnp.float32),
        pltpu.VMEM((1, 2, CHUNK), jnp.int32),
        pltpu.VMEM((1, CHUNK), jnp.float32),
        pltpu.VMEM((1, CHUNK, D), jnp.float32),
        pltpu.VMEM((640,), jnp.float32),
        pltpu.VMEM_SHARED((NPAD, D), jnp.float32),
        pltpu.VMEM_SHARED((NPAD,), jnp.float32),
        pltpu.SemaphoreType.DMA,
    ],
)


def _merge_body(acc_ref, den_ref, ind_ref, o_ref):
  den = jnp.maximum(den_ref[0] + den_ref[1], 1e-16)
  ind = jnp.maximum(ind_ref[0] + ind_ref[1], 1.0)
  h = jnp.sqrt(ind) / den
  o_ref[...] = (acc_ref[0] + acc_ref[1]) * h[:, None]


def kernel(x, edge_index, W, attn_l, attn_r):
  al = attn_l.reshape(1, D)
  ar = attn_r.reshape(1, D)

  outdeg, indeg = _deg_call(edge_index)
  outdeg = outdeg.reshape(NC, NPAD)

  nblocks = NPAD // CHUNK
  feat_src, elr = pl.pallas_call(
      _proj_body,
      grid=(nblocks,),
      in_specs=[
          pl.BlockSpec((CHUNK, D), lambda i: (i, 0)),
          pl.BlockSpec((D, D), lambda i: (0, 0)),
          pl.BlockSpec((1, D), lambda i: (0, 0)),
          pl.BlockSpec((1, D), lambda i: (0, 0)),
          pl.BlockSpec((NC, CHUNK), lambda i: (0, i)),
      ],
      out_specs=[
          pl.BlockSpec((CHUNK, D), lambda i: (i, 0)),
          pl.BlockSpec((NC, CHUNK), lambda i: (0, i)),
      ],
      out_shape=[
          jax.ShapeDtypeStruct((N, D), jnp.float32),
          jax.ShapeDtypeStruct((NC, NPAD), jnp.float32),
      ],
  )(x, W, al, ar, outdeg)

  acc, den = _edge_call(edge_index, feat_src, elr.reshape(NC * NPAD))
  den = den.reshape(NC, NPAD)
  ind = indeg.reshape(NC, NPAD)

  rst = pl.pallas_call(
      _merge_body,
      grid=(nblocks,),
      in_specs=[
          pl.BlockSpec((NC, CHUNK, D), lambda i: (0, i, 0)),
          pl.BlockSpec((NC, CHUNK), lambda i: (0, i)),
          pl.BlockSpec((NC, CHUNK), lambda i: (0, i)),
      ],
      out_specs=pl.BlockSpec((CHUNK, D), lambda i: (i, 0)),
      out_shape=jax.ShapeDtypeStruct((N, D), jnp.float32),
  )(acc, den, ind)

  return rst.reshape(N, 1, OUT)
